# baseline (speedup 1.0000x reference)
import functools

import jax
import jax.numpy as jnp
from jax import lax
from jax.experimental import pallas as pl
from jax.experimental.pallas import tpu as pltpu
from jax.experimental.pallas import tpu_sc as plsc

_B = 128
_CCH = 16


def _routing_metadata(top_k_indices, top_k_weights, E, B, NB):
    T, K = top_k_indices.shape
    TK = T * K
    NR = NB * B
    i32 = jnp.int32
    e_flat = top_k_indices.astype(i32).reshape(TK)
    w_flat = top_k_weights.reshape(TK)
    onehot = (e_flat[:, None] == jnp.arange(E, dtype=i32)[None, :]).astype(i32)
    csum = jnp.cumsum(onehot, axis=0)
    counts = csum[TK - 1]
    occ = jnp.take_along_axis(csum, e_flat[:, None], axis=1)[:, 0] - 1
    bpe = (counts + B - 1) // B
    pad_off = (jnp.cumsum(bpe) - bpe) * B
    P = (pad_off[e_flat] + occ).astype(i32)
    src_tokens = jnp.zeros(NR, i32).at[P].set(jnp.arange(TK, dtype=i32) // K)
    w_pad = jnp.zeros(NR, jnp.float32).at[P].set(w_flat)
    pos = P.reshape(T, K)
    nb_used = jnp.sum(bpe).astype(i32)
    be = jnp.searchsorted(jnp.cumsum(bpe), jnp.arange(NB, dtype=i32), side="right")
    last_e = jnp.max(jnp.where(counts > 0, jnp.arange(E, dtype=i32), 0))
    block_expert = jnp.where(jnp.arange(NB) < nb_used,
                             jnp.minimum(be.astype(i32), E - 1), last_e)
    first_blk = jnp.concatenate([
        jnp.ones(1, i32),
        (block_expert[1:] != block_expert[:-1]).astype(i32)])
    grp = jnp.cumsum(first_blk).astype(i32) - 1
    ng = jnp.sum(first_blk).astype(i32)
    gexp = jnp.zeros(NB, i32).at[grp].set(block_expert)
    nxt_exp = gexp[jnp.minimum(grp + 1, NB - 1)]
    lims = jnp.stack([nb_used, ng])
    return (src_tokens, w_pad, pos, block_expert, first_blk, grp, nxt_exp, lims)


def _tc_grouped_mlp(hidden, gate_up_proj, down_proj, w_pad, src_tokens,
                    block_expert, first_blk, grp, nxt_exp, lims, NB, B, D, I):
    def body(idx_ref, be_ref, fb_ref, grp_ref, ne_ref, lims_ref, hid_hbm,
             wgu_hbm, wd_hbm, w_ref, out_ref, xbuf, wgu_buf, wd_buf,
             semx, semg, semd):
        b = pl.program_id(0)
        nb = lims_ref[0]
        ng = lims_ref[1]
        par = lax.rem(grp_ref[b], 2)
        xpar = lax.rem(b, 2)

        def issue_rows(blk, bpar):
            def row(r, _):
                t = idx_ref[blk * B + r]
                pltpu.make_async_copy(hid_hbm.at[t], xbuf.at[bpar, r],
                                      semx.at[bpar]).start()
                return 0
            lax.fori_loop(0, B, row, 0)

        GC, DC = 4, 2
        gch, dch = (2 * I) // GC, D // DC

        def w_copies(eid, p):
            cps = []
            for j in range(GC):
                cps.append(pltpu.make_async_copy(
                    wgu_hbm.at[eid, pl.ds(j * gch, gch)],
                    wgu_buf.at[p, pl.ds(j * gch, gch)], semg.at[p, j]))
            for j in range(DC):
                cps.append(pltpu.make_async_copy(
                    wd_hbm.at[eid, pl.ds(j * dch, dch)],
                    wd_buf.at[p, pl.ds(j * dch, dch)], semd.at[p, j]))
            return cps

        @pl.when(b == 0)
        def _():
            for cp in w_copies(be_ref[0], 0):
                cp.start()
            issue_rows(0, 0)

        @pl.when(fb_ref[b] == 1)
        def _():
            nxt = grp_ref[b] + 1
            npar = lax.rem(nxt, 2)

            @pl.when(nxt < ng)
            def _():
                for cp in w_copies(ne_ref[b], npar):
                    cp.start()

            for cp in w_copies(be_ref[b], par):
                cp.wait()

        @pl.when(b < nb)
        def _():
            @pl.when(b + 1 < nb)
            def _():
                issue_rows(b + 1, 1 - xpar)

            pltpu.make_async_copy(hid_hbm.at[pl.ds(0, B)], xbuf.at[xpar],
                                  semx.at[xpar]).wait()
            x = xbuf[xpar]
            gu = lax.dot_general(x, wgu_buf[par], (((1,), (1,)), ((), ())),
                                 preferred_element_type=jnp.float32)
            gate = gu[:, :I]
            up = gu[:, I:]
            y = gate * jax.nn.sigmoid(gate) * up
            o = lax.dot_general(y, wd_buf[par], (((1,), (1,)), ((), ())),
                                preferred_element_type=jnp.float32)
            out_ref[...] = o * w_ref[0, 0, :][:, None]

    grid_spec = pltpu.PrefetchScalarGridSpec(
        num_scalar_prefetch=6,
        grid=(NB,),
        in_specs=[
            pl.BlockSpec(memory_space=pl.ANY),
            pl.BlockSpec(memory_space=pl.ANY),
            pl.BlockSpec(memory_space=pl.ANY),
            pl.BlockSpec((1, 1, B), lambda b, *_: (b, 0, 0)),
        ],
        out_specs=pl.BlockSpec((B, D), lambda b, *_: (b, 0)),
        scratch_shapes=[
            pltpu.VMEM((2, B, D), jnp.float32),
            pltpu.VMEM((2, 2 * I, D), jnp.float32),
            pltpu.VMEM((2, D, I), jnp.float32),
            pltpu.SemaphoreType.DMA((2,)),
            pltpu.SemaphoreType.DMA((2, 4)),
            pltpu.SemaphoreType.DMA((2, 2)),
        ],
    )
    return pl.pallas_call(
        body,
        grid_spec=grid_spec,
        out_shape=jax.ShapeDtypeStruct((NB * B, D), jnp.float32),
    )(src_tokens, block_expert, first_blk, grp, nxt_exp, lims, hidden,
      gate_up_proj, down_proj, w_pad.reshape(NB, 1, B))


def _sc_combine(ys, pos, T, D, K):
    info = plsc.get_sparse_core_info()
    NW = info.num_cores * info.num_subcores
    t_per_w = T // NW
    n_ch = t_per_w // _CCH
    n_sl = D // info.num_lanes
    L = info.num_lanes
    mesh = plsc.VectorSubcoreMesh(core_axis_name="c", subcore_axis_name="s")
    pos_cols = [pos[:, k] for k in range(K)]
    assert K == 2

    @functools.partial(
        pl.kernel, mesh=mesh,
        out_type=jax.ShapeDtypeStruct((T, D), jnp.float32),
        scratch_types=[
            pltpu.VMEM((_CCH,), jnp.int32),
            pltpu.VMEM((_CCH,), jnp.int32),
            pltpu.VMEM((_CCH,), jnp.int32),
            pltpu.VMEM((_CCH,), jnp.int32),
            pltpu.VMEM((_CCH, D), jnp.float32),
            pltpu.VMEM((_CCH, D), jnp.float32),
            pltpu.VMEM((_CCH, D), jnp.float32),
            pltpu.VMEM((_CCH, D), jnp.float32),
            pltpu.SemaphoreType.DMA,
            pltpu.SemaphoreType.DMA,
            pltpu.SemaphoreType.DMA,
            pltpu.SemaphoreType.DMA,
            pltpu.SemaphoreType.DMA,
            pltpu.SemaphoreType.DMA,
        ],
    )
    def k(ys_hbm, p0_hbm, p1_hbm, out_hbm, ia0, ia1, ib0, ib1,
          a0, a1, b0, b1, sa0, sa1, sb0, sb1, sw0, sw1):
        wid = lax.axis_index("s") * info.num_cores + lax.axis_index("c")
        base = wid * t_per_w
        ia, ib = [ia0, ia1], [ib0, ib1]
        av, bv = [a0, a1], [b0, b1]
        sa, sb, sw = [sa0, sa1], [sb0, sb1], [sw0, sw1]
        gha = [None, None]
        ghb = [None, None]
        wh = [None, None]

        def start(c):
            p = c % 2
            lo = base + c * _CCH
            pltpu.sync_copy(p0_hbm.at[pl.ds(lo, _CCH)], ia[p])
            gha[p] = pltpu.async_copy(ys_hbm.at[ia[p]], av[p], sa[p])
            pltpu.sync_copy(p1_hbm.at[pl.ds(lo, _CCH)], ib[p])
            ghb[p] = pltpu.async_copy(ys_hbm.at[ib[p]], bv[p], sb[p])

        start(0)
        for c in range(n_ch):
            p, q = c % 2, (c + 1) % 2
            if c + 1 < n_ch:
                if wh[q] is not None:
                    wh[q].wait()
                start(c + 1)
            gha[p].wait()
            ghb[p].wait()
            x, y = av[p], bv[p]

            def row_add(r, _):
                for col in range(n_sl):
                    sl = pl.ds(col * L, L)
                    x[r, sl] = x[r, sl] + y[r, sl]
                return 0

            lax.fori_loop(0, _CCH, row_add, 0)
            wh[p] = pltpu.async_copy(
                x, out_hbm.at[pl.ds(base + c * _CCH, _CCH)], sw[p])
        for h in wh:
            if h is not None:
                h.wait()

    return k(ys, *pos_cols)


def kernel(hidden_states, top_k_indices, top_k_weights, gate_up_proj, down_proj):
    T, D = hidden_states.shape
    K = top_k_indices.shape[1]
    E, I2, _ = gate_up_proj.shape
    I = I2 // 2
    TK = T * K
    B = _B
    NB = -(-TK // B) + E - 1
    NB = -(-NB // 2) * 2

    (src_tokens, w_pad, pos, block_expert, first_blk, grp, nxt_exp,
     lims) = _routing_metadata(top_k_indices, top_k_weights, E, B, NB)

    ys = _tc_grouped_mlp(hidden_states, gate_up_proj, down_proj, w_pad,
                         src_tokens, block_expert, first_blk, grp, nxt_exp,
                         lims, NB, B, D, I)
    return _sc_combine(ys, pos, T, D, K)


# speedup vs baseline: 1.0402x; 1.0402x over previous
import functools

import jax
import jax.numpy as jnp
from jax import lax
from jax.experimental import pallas as pl
from jax.experimental.pallas import tpu as pltpu
from jax.experimental.pallas import tpu_sc as plsc

_B = 128
_CCH = 16


def _routing_metadata(top_k_indices, top_k_weights, E, B, NB):
    T, K = top_k_indices.shape
    TK = T * K
    NR = NB * B
    i32 = jnp.int32
    e_flat = top_k_indices.astype(i32).reshape(TK)
    w_flat = top_k_weights.reshape(TK)
    onehot = (e_flat[:, None] == jnp.arange(E, dtype=i32)[None, :]).astype(i32)
    csum = jnp.cumsum(onehot, axis=0)
    counts = csum[TK - 1]
    occ = jnp.take_along_axis(csum, e_flat[:, None], axis=1)[:, 0] - 1
    bpe = (counts + B - 1) // B
    pad_off = (jnp.cumsum(bpe) - bpe) * B
    P = (pad_off[e_flat] + occ).astype(i32)
    src_tokens = jnp.zeros(NR, i32).at[P].set(jnp.arange(TK, dtype=i32) // K)
    w_pad = jnp.zeros(NR, jnp.float32).at[P].set(w_flat)
    pos = P.reshape(T, K)
    nb_used = jnp.sum(bpe).astype(i32)
    be = jnp.searchsorted(jnp.cumsum(bpe), jnp.arange(NB, dtype=i32), side="right")
    last_e = jnp.max(jnp.where(counts > 0, jnp.arange(E, dtype=i32), 0))
    block_expert = jnp.where(jnp.arange(NB) < nb_used,
                             jnp.minimum(be.astype(i32), E - 1), last_e)
    first_blk = jnp.concatenate([
        jnp.ones(1, i32),
        (block_expert[1:] != block_expert[:-1]).astype(i32)])
    grp = jnp.cumsum(first_blk).astype(i32) - 1
    ng = jnp.sum(first_blk).astype(i32)
    gexp = jnp.zeros(NB, i32).at[grp].set(block_expert)
    nxt_exp = gexp[jnp.minimum(grp + 1, NB - 1)]
    lims = jnp.stack([nb_used, ng])
    return (src_tokens, w_pad, pos, block_expert, first_blk, grp, nxt_exp, lims)


def _tc_grouped_mlp(hidden, gate_up_proj, down_proj, w_pad, src_tokens,
                    block_expert, first_blk, grp, nxt_exp, lims, NB, B, D, I):
    def body(idx_ref, be_ref, fb_ref, grp_ref, ne_ref, lims_ref, hid_hbm,
             wgu_hbm, wd_hbm, w_ref, out_ref, xbuf, wgu_buf, wd_buf,
             semx, semg, semd):
        b = pl.program_id(0)
        nb = lims_ref[0]
        ng = lims_ref[1]
        par = lax.rem(grp_ref[b], 2)
        xpar = lax.rem(b, 3)

        def issue_rows(blk, bpar):
            def row(r, _):
                t = idx_ref[blk * B + r]
                pltpu.make_async_copy(hid_hbm.at[t], xbuf.at[bpar, r],
                                      semx.at[bpar]).start()
                return 0
            lax.fori_loop(0, B, row, 0)

        @pl.when(b == 0)
        def _():
            pltpu.make_async_copy(wgu_hbm.at[be_ref[0]], wgu_buf.at[0],
                                  semg.at[0]).start()
            pltpu.make_async_copy(wd_hbm.at[be_ref[0]], wd_buf.at[0],
                                  semd.at[0]).start()
            issue_rows(0, 0)

            @pl.when(1 < nb)
            def _():
                issue_rows(1, 1)

        @pl.when(fb_ref[b] == 1)
        def _():
            nxt = grp_ref[b] + 1
            npar = lax.rem(nxt, 2)

            @pl.when(nxt < ng)
            def _():
                pltpu.make_async_copy(wgu_hbm.at[ne_ref[b]],
                                      wgu_buf.at[npar], semg.at[npar]).start()
                pltpu.make_async_copy(wd_hbm.at[ne_ref[b]],
                                      wd_buf.at[npar], semd.at[npar]).start()

            pltpu.make_async_copy(wgu_hbm.at[be_ref[b]], wgu_buf.at[par],
                                  semg.at[par]).wait()
            pltpu.make_async_copy(wd_hbm.at[be_ref[b]], wd_buf.at[par],
                                  semd.at[par]).wait()

        @pl.when(b < nb)
        def _():
            @pl.when(b + 2 < nb)
            def _():
                issue_rows(b + 2, lax.rem(b + 2, 3))

            pltpu.make_async_copy(hid_hbm.at[pl.ds(0, B)], xbuf.at[xpar],
                                  semx.at[xpar]).wait()
            x = xbuf[xpar]
            gu = lax.dot_general(x, wgu_buf[par], (((1,), (1,)), ((), ())),
                                 preferred_element_type=jnp.float32)
            gate = gu[:, :I]
            up = gu[:, I:]
            y = gate * jax.nn.sigmoid(gate) * up
            o = lax.dot_general(y, wd_buf[par], (((1,), (1,)), ((), ())),
                                preferred_element_type=jnp.float32)
            out_ref[...] = o * w_ref[0, 0, :][:, None]

    grid_spec = pltpu.PrefetchScalarGridSpec(
        num_scalar_prefetch=6,
        grid=(NB,),
        in_specs=[
            pl.BlockSpec(memory_space=pl.ANY),
            pl.BlockSpec(memory_space=pl.ANY),
            pl.BlockSpec(memory_space=pl.ANY),
            pl.BlockSpec((1, 1, B), lambda b, *_: (b, 0, 0)),
        ],
        out_specs=pl.BlockSpec((B, D), lambda b, *_: (b, 0)),
        scratch_shapes=[
            pltpu.VMEM((3, B, D), jnp.float32),
            pltpu.VMEM((2, 2 * I, D), jnp.float32),
            pltpu.VMEM((2, D, I), jnp.float32),
            pltpu.SemaphoreType.DMA((3,)),
            pltpu.SemaphoreType.DMA((2,)),
            pltpu.SemaphoreType.DMA((2,)),
        ],
    )
    return pl.pallas_call(
        body,
        grid_spec=grid_spec,
        out_shape=jax.ShapeDtypeStruct((NB * B, D), jnp.float32),
    )(src_tokens, block_expert, first_blk, grp, nxt_exp, lims, hidden,
      gate_up_proj, down_proj, w_pad.reshape(NB, 1, B))


def _sc_combine(ys, pos, T, D, K):
    info = plsc.get_sparse_core_info()
    NW = info.num_cores * info.num_subcores
    t_per_w = T // NW
    n_ch = t_per_w // _CCH
    n_sl = D // info.num_lanes
    L = info.num_lanes
    mesh = plsc.VectorSubcoreMesh(core_axis_name="c", subcore_axis_name="s")
    pos_cols = [pos[:, k] for k in range(K)]
    assert K == 2

    @functools.partial(
        pl.kernel, mesh=mesh,
        out_type=jax.ShapeDtypeStruct((T, D), jnp.float32),
        scratch_types=[
            pltpu.VMEM((_CCH,), jnp.int32),
            pltpu.VMEM((_CCH,), jnp.int32),
            pltpu.VMEM((_CCH,), jnp.int32),
            pltpu.VMEM((_CCH,), jnp.int32),
            pltpu.VMEM((_CCH, D), jnp.float32),
            pltpu.VMEM((_CCH, D), jnp.float32),
            pltpu.VMEM((_CCH, D), jnp.float32),
            pltpu.VMEM((_CCH, D), jnp.float32),
            pltpu.SemaphoreType.DMA,
            pltpu.SemaphoreType.DMA,
            pltpu.SemaphoreType.DMA,
            pltpu.SemaphoreType.DMA,
            pltpu.SemaphoreType.DMA,
            pltpu.SemaphoreType.DMA,
        ],
    )
    def k(ys_hbm, p0_hbm, p1_hbm, out_hbm, ia0, ia1, ib0, ib1,
          a0, a1, b0, b1, sa0, sa1, sb0, sb1, sw0, sw1):
        wid = lax.axis_index("s") * info.num_cores + lax.axis_index("c")
        base = wid * t_per_w
        ia, ib = [ia0, ia1], [ib0, ib1]
        av, bv = [a0, a1], [b0, b1]
        sa, sb, sw = [sa0, sa1], [sb0, sb1], [sw0, sw1]
        gha = [None, None]
        ghb = [None, None]
        wh = [None, None]

        def start(c):
            p = c % 2
            lo = base + c * _CCH
            pltpu.sync_copy(p0_hbm.at[pl.ds(lo, _CCH)], ia[p])
            gha[p] = pltpu.async_copy(ys_hbm.at[ia[p]], av[p], sa[p])
            pltpu.sync_copy(p1_hbm.at[pl.ds(lo, _CCH)], ib[p])
            ghb[p] = pltpu.async_copy(ys_hbm.at[ib[p]], bv[p], sb[p])

        start(0)
        for c in range(n_ch):
            p, q = c % 2, (c + 1) % 2
            if c + 1 < n_ch:
                if wh[q] is not None:
                    wh[q].wait()
                start(c + 1)
            gha[p].wait()
            ghb[p].wait()
            x, y = av[p], bv[p]

            def row_add(r, _):
                for col in range(n_sl):
                    sl = pl.ds(col * L, L)
                    x[r, sl] = x[r, sl] + y[r, sl]
                return 0

            lax.fori_loop(0, _CCH, row_add, 0)
            wh[p] = pltpu.async_copy(
                x, out_hbm.at[pl.ds(base + c * _CCH, _CCH)], sw[p])
        for h in wh:
            if h is not None:
                h.wait()

    return k(ys, *pos_cols)


def kernel(hidden_states, top_k_indices, top_k_weights, gate_up_proj, down_proj):
    T, D = hidden_states.shape
    K = top_k_indices.shape[1]
    E, I2, _ = gate_up_proj.shape
    I = I2 // 2
    TK = T * K
    B = _B
    NB = -(-TK // B) + E - 1
    NB = -(-NB // 2) * 2

    (src_tokens, w_pad, pos, block_expert, first_blk, grp, nxt_exp,
     lims) = _routing_metadata(top_k_indices, top_k_weights, E, B, NB)

    ys = _tc_grouped_mlp(hidden_states, gate_up_proj, down_proj, w_pad,
                         src_tokens, block_expert, first_blk, grp, nxt_exp,
                         lims, NB, B, D, I)
    return _sc_combine(ys, pos, T, D, K)
